# baseline (speedup 1.0000x reference)
import functools

import jax
import jax.numpy as jnp
from jax import lax
from jax.experimental import pallas as pl
from jax.experimental.pallas import tpu as pltpu
from jax.experimental.pallas import tpu_sc as plsc

N_NODES = 10000
T_STEPS = 12
C_IN = 128
HID = 128
C_CLS = 10
N_EDGES = 320000

NUM_CORES = 2
NUM_SUBCORES = 16
NUM_WORKERS = NUM_CORES * NUM_SUBCORES
EDGES_PER_WORKER = N_EDGES // NUM_WORKERS
CHUNK = 80
NUM_CHUNKS = EDGES_PER_WORKER // CHUNK
NUM_GRPS = 5
GRP_CHUNKS = NUM_CHUNKS // NUM_GRPS
N_PAD = 10240
ROWS_PER_TILE = N_PAD // NUM_SUBCORES
ZROWS = 64


def _sc_agg_body(x_hbm, gidx_hbm, cidx_hbm, dst_hbm, zeros_hbm, ones_hbm,
                 agg_hbm, cnt_hbm,
                 gidx_v, dst_v, rows_a, rows_b, rows_c, acc_sh,
                 sem_a, sem_b, sem_c, ssem_a, ssem_b, ssem_c):
    c = lax.axis_index("c")
    s = lax.axis_index("s")
    wid = s * NUM_CORES + c
    r0 = s * ROWS_PER_TILE

    def zero_my_rows():
        pltpu.sync_copy(zeros_hbm, acc_sh.at[pl.ds(r0, ROWS_PER_TILE)])

    zero_my_rows()
    plsc.subcore_barrier()

    def run_pass(idx_hbm_slab, table_hbm):
        def wait_g(q, rows_x, gsem):
            pltpu.make_async_copy(table_hbm.at[gidx_v.at[q]],
                                  rows_x, gsem).wait()

        def scat(q, rows_x, ssem):
            pltpu.async_copy(rows_x, acc_sh.at[dst_v.at[q]], ssem, add=True)

        def wait_s(q, rows_x, ssem):
            pltpu.make_async_copy(rows_x, acc_sh.at[dst_v.at[q]], ssem).wait()

        def gath(q, rows_x, gsem):
            pltpu.async_copy(table_hbm.at[gidx_v.at[q]], rows_x, gsem)

        for g in range(NUM_GRPS):
            pltpu.sync_copy(idx_hbm_slab.at[g], gidx_v)
            pltpu.sync_copy(dst_hbm.at[wid, g], dst_v)
            gath(0, rows_a, sem_a)
            gath(1, rows_b, sem_b)
            wait_g(0, rows_a, sem_a); scat(0, rows_a, ssem_a)
            gath(2, rows_c, sem_c)
            wait_g(1, rows_b, sem_b); scat(1, rows_b, ssem_b)
            wait_s(0, rows_a, ssem_a)
            gath(3, rows_a, sem_a)
            wait_g(2, rows_c, sem_c); scat(2, rows_c, ssem_c)
            wait_s(1, rows_b, ssem_b)
            gath(4, rows_b, sem_b)

            def tri_body(p, carry):
                b = 3 * p
                wait_g(b, rows_a, sem_a); scat(b, rows_a, ssem_a)
                wait_s(b - 1, rows_c, ssem_c)
                gath(b + 2, rows_c, sem_c)
                wait_g(b + 1, rows_b, sem_b); scat(b + 1, rows_b, ssem_b)
                wait_s(b, rows_a, ssem_a)
                gath(b + 3, rows_a, sem_a)
                wait_g(b + 2, rows_c, sem_c); scat(b + 2, rows_c, ssem_c)
                wait_s(b + 1, rows_b, ssem_b)
                gath(b + 4, rows_b, sem_b)
                return carry
            lax.fori_loop(1, 7, tri_body, 0)
            wait_g(21, rows_a, sem_a); scat(21, rows_a, ssem_a)
            wait_s(20, rows_c, ssem_c)
            gath(23, rows_c, sem_c)
            wait_g(22, rows_b, sem_b); scat(22, rows_b, ssem_b)
            wait_s(21, rows_a, ssem_a)
            gath(24, rows_a, sem_a)
            wait_g(23, rows_c, sem_c); scat(23, rows_c, ssem_c)
            wait_s(22, rows_b, ssem_b)
            wait_g(24, rows_a, sem_a); scat(24, rows_a, ssem_a)
            wait_s(23, rows_c, ssem_c)
            wait_s(24, rows_a, ssem_a)

    run_pass(cidx_hbm.at[wid], ones_hbm)
    plsc.subcore_barrier()
    pltpu.sync_copy(acc_sh.at[pl.ds(r0, ROWS_PER_TILE)],
                    cnt_hbm.at[c, pl.ds(r0, ROWS_PER_TILE)])
    zero_my_rows()
    plsc.subcore_barrier()

    def t_body(t, carry):
        run_pass(gidx_hbm.at[t, wid], x_hbm)
        plsc.subcore_barrier()
        pltpu.sync_copy(acc_sh.at[pl.ds(r0, ROWS_PER_TILE)],
                        agg_hbm.at[c, t, pl.ds(r0, ROWS_PER_TILE)])
        zero_my_rows()
        plsc.subcore_barrier()
        return carry
    lax.fori_loop(0, T_STEPS, t_body, 0)


@jax.jit
def _sc_aggregate(x_flat, gidx, cidx, dst, zeros, ones):
    mesh = plsc.VectorSubcoreMesh(core_axis_name="c", subcore_axis_name="s")
    run = pl.kernel(
        _sc_agg_body,
        out_type=[
            jax.ShapeDtypeStruct((NUM_CORES, T_STEPS, N_PAD, C_IN),
                                 jnp.float32),
            jax.ShapeDtypeStruct((NUM_CORES, N_PAD, C_IN), jnp.float32),
        ],
        mesh=mesh,
        scratch_types=[
            pltpu.VMEM((GRP_CHUNKS, CHUNK), jnp.int32),
            pltpu.VMEM((GRP_CHUNKS, CHUNK), jnp.int32),
            pltpu.VMEM((CHUNK, C_IN), jnp.float32),
            pltpu.VMEM((CHUNK, C_IN), jnp.float32),
            pltpu.VMEM((CHUNK, C_IN), jnp.float32),
            pltpu.VMEM_SHARED((N_PAD, C_IN), jnp.float32),
            pltpu.SemaphoreType.DMA,
            pltpu.SemaphoreType.DMA,
            pltpu.SemaphoreType.DMA,
            pltpu.SemaphoreType.DMA,
            pltpu.SemaphoreType.DMA,
            pltpu.SemaphoreType.DMA,
        ],
    )
    return run(x_flat, gidx, cidx, dst, zeros, ones)


def _tc_dense_body(x_ref, agg_ref, cnt_ref, wl_ref, bl_ref, wr_ref,
                   wih_ref, whh_ref, bih_ref, bhh_ref,
                   wcls_ref, bcls_ref, wrec_ref, brec_ref,
                   logits_ref, recon_ref):
    cnt = cnt_ref[0, :, 0:1] + cnt_ref[1, :, 0:1]
    rdenom = 1.0 / jnp.maximum(cnt, 1.0)
    nb = x_ref.shape[0]
    h = jnp.zeros((nb, HID), jnp.float32)
    for t in range(T_STEPS):
        x_t = x_ref[:, t, :]
        mean = (agg_ref[0, t] + agg_ref[1, t]) * rdenom
        sp = jnp.maximum(
            jnp.dot(mean, wl_ref[...], preferred_element_type=jnp.float32)
            + bl_ref[...]
            + jnp.dot(x_t, wr_ref[...], preferred_element_type=jnp.float32),
            0.0)
        gi = jnp.dot(sp, wih_ref[...],
                     preferred_element_type=jnp.float32) + bih_ref[...]
        gh = jnp.dot(h, whh_ref[...],
                     preferred_element_type=jnp.float32) + bhh_ref[...]
        r = jax.nn.sigmoid(gi[:, :HID] + gh[:, :HID])
        z = jax.nn.sigmoid(gi[:, HID:2 * HID] + gh[:, HID:2 * HID])
        ng = jnp.tanh(gi[:, 2 * HID:] + r * gh[:, 2 * HID:])
        h = (1.0 - z) * ng + z * h
        logits_ref[:, t, :] = jnp.dot(
            h, wcls_ref[...], preferred_element_type=jnp.float32) + bcls_ref[...]
        recon_ref[:, t, :] = jnp.dot(
            h, wrec_ref[...], preferred_element_type=jnp.float32) + brec_ref[...]


@functools.partial(jax.jit, static_argnames=("nb", "interpret"))
def _tc_dense(x, agg, cnt, wl_t, bl2, wr_t, wih_t, whh_t, bih2, bhh2,
              wcls_t, bcls2, wrec_t, brec2, nb=400, interpret=False):
    n = x.shape[0]
    grid = (n // nb,)
    full = lambda shape: pl.BlockSpec(shape, lambda i: tuple(0 for _ in shape))
    return pl.pallas_call(
        _tc_dense_body,
        grid=grid,
        in_specs=[
            pl.BlockSpec((nb, T_STEPS, C_IN), lambda i: (i, 0, 0)),
            pl.BlockSpec((NUM_CORES, T_STEPS, nb, C_IN),
                         lambda i: (0, 0, i, 0)),
            pl.BlockSpec((NUM_CORES, nb, C_IN), lambda i: (0, i, 0)),
            full((C_IN, HID)), full((1, HID)), full((C_IN, HID)),
            full((HID, 3 * HID)), full((HID, 3 * HID)),
            full((1, 3 * HID)), full((1, 3 * HID)),
            full((HID, C_CLS)), full((1, C_CLS)),
            full((HID, C_IN)), full((1, C_IN)),
        ],
        out_specs=[
            pl.BlockSpec((nb, T_STEPS, C_CLS), lambda i: (i, 0, 0)),
            pl.BlockSpec((nb, T_STEPS, C_IN), lambda i: (i, 0, 0)),
        ],
        out_shape=[
            jax.ShapeDtypeStruct((n, T_STEPS, C_CLS), jnp.float32),
            jax.ShapeDtypeStruct((n, T_STEPS, C_IN), jnp.float32),
        ],
        interpret=interpret,
    )(x, agg, cnt, wl_t, bl2, wr_t, wih_t, whh_t, bih2, bhh2,
      wcls_t, bcls2, wrec_t, brec2)


def kernel(x, edge_index, Wl, bl, Wr, W_ih, W_hh, b_ih, b_hh,
           W_cls, b_cls, W_rec, b_rec):
    src = edge_index[0]
    dst = edge_index[1]
    gidx = (src[None, :] * T_STEPS
            + jnp.arange(T_STEPS, dtype=jnp.int32)[:, None])
    gidx = gidx.reshape(T_STEPS, NUM_WORKERS, NUM_GRPS, GRP_CHUNKS, CHUNK)
    cidx = jnp.remainder(jnp.arange(N_EDGES, dtype=jnp.int32), 128)
    cidx = cidx.reshape(NUM_WORKERS, NUM_GRPS, GRP_CHUNKS, CHUNK)
    dst_r = dst.reshape(NUM_WORKERS, NUM_GRPS, GRP_CHUNKS, CHUNK)
    x_flat = x.reshape(N_NODES * T_STEPS, C_IN)
    zeros = jnp.zeros((ROWS_PER_TILE, C_IN), jnp.float32)
    ones = jnp.ones((128, C_IN), jnp.float32)
    agg, cnt = _sc_aggregate(x_flat, gidx, cidx, dst_r, zeros, ones)
    logits, recon = _tc_dense(
        x, agg, cnt,
        Wl.T, bl[None, :], Wr.T,
        W_ih.T, W_hh.T, b_ih[None, :], b_hh[None, :],
        W_cls.T, b_cls[None, :], W_rec.T, b_rec[None, :])
    return (logits, recon)


# speedup vs baseline: 1.0400x; 1.0400x over previous
import functools

import jax
import jax.numpy as jnp
from jax import lax
from jax.experimental import pallas as pl
from jax.experimental.pallas import tpu as pltpu
from jax.experimental.pallas import tpu_sc as plsc

N_NODES = 10000
T_STEPS = 12
C_IN = 128
HID = 128
C_CLS = 10
N_EDGES = 320000

NUM_CORES = 2
NUM_SUBCORES = 16
NUM_WORKERS = NUM_CORES * NUM_SUBCORES
EDGES_PER_WORKER = N_EDGES // NUM_WORKERS
CHUNK = 80
NUM_CHUNKS = EDGES_PER_WORKER // CHUNK
NUM_GRPS = 5
GRP_CHUNKS = NUM_CHUNKS // NUM_GRPS
N_PAD = 10240
ROWS_PER_TILE = N_PAD // NUM_SUBCORES
ZROWS = 64


def _sc_agg_body(x_hbm, gidx_hbm, cidx_hbm, dst_hbm, zeros_hbm, ones_hbm,
                 agg_hbm, cnt_hbm,
                 gidx_v, dst_v, rows_a, rows_b, rows_c, acc_sh,
                 sem_a, sem_b, sem_c):
    c = lax.axis_index("c")
    s = lax.axis_index("s")
    wid = s * NUM_CORES + c
    r0 = s * ROWS_PER_TILE

    def zero_my_rows():
        pltpu.sync_copy(zeros_hbm, acc_sh.at[pl.ds(r0, ROWS_PER_TILE)])

    zero_my_rows()
    plsc.subcore_barrier()

    def run_pass(idx_hbm_slab, table_hbm):
        for g in range(NUM_GRPS):
            pltpu.sync_copy(idx_hbm_slab.at[g], gidx_v)
            pltpu.sync_copy(dst_hbm.at[wid, g], dst_v)
            pltpu.async_copy(table_hbm.at[gidx_v.at[0]], rows_a, sem_a)
            pltpu.async_copy(table_hbm.at[gidx_v.at[1]], rows_b, sem_b)

            def tri_body(p, carry):
                b = 3 * p
                pltpu.async_copy(table_hbm.at[gidx_v.at[b + 2]],
                                 rows_c, sem_c)
                pltpu.make_async_copy(table_hbm.at[gidx_v.at[b]],
                                      rows_a, sem_a).wait()
                pltpu.sync_copy(rows_a, acc_sh.at[dst_v.at[b]], add=True)
                pltpu.async_copy(table_hbm.at[gidx_v.at[b + 3]],
                                 rows_a, sem_a)
                pltpu.make_async_copy(table_hbm.at[gidx_v.at[b + 1]],
                                      rows_b, sem_b).wait()
                pltpu.sync_copy(rows_b, acc_sh.at[dst_v.at[b + 1]], add=True)
                pltpu.async_copy(table_hbm.at[gidx_v.at[b + 4]],
                                 rows_b, sem_b)
                pltpu.make_async_copy(table_hbm.at[gidx_v.at[b + 2]],
                                      rows_c, sem_c).wait()
                pltpu.sync_copy(rows_c, acc_sh.at[dst_v.at[b + 2]], add=True)
                return carry
            lax.fori_loop(0, 7, tri_body, 0)
            pltpu.async_copy(table_hbm.at[gidx_v.at[23]], rows_c, sem_c)
            pltpu.make_async_copy(table_hbm.at[gidx_v.at[21]],
                                  rows_a, sem_a).wait()
            pltpu.sync_copy(rows_a, acc_sh.at[dst_v.at[21]], add=True)
            pltpu.async_copy(table_hbm.at[gidx_v.at[24]], rows_a, sem_a)
            pltpu.make_async_copy(table_hbm.at[gidx_v.at[22]],
                                  rows_b, sem_b).wait()
            pltpu.sync_copy(rows_b, acc_sh.at[dst_v.at[22]], add=True)
            pltpu.make_async_copy(table_hbm.at[gidx_v.at[23]],
                                  rows_c, sem_c).wait()
            pltpu.sync_copy(rows_c, acc_sh.at[dst_v.at[23]], add=True)
            pltpu.make_async_copy(table_hbm.at[gidx_v.at[24]],
                                  rows_a, sem_a).wait()
            pltpu.sync_copy(rows_a, acc_sh.at[dst_v.at[24]], add=True)

    run_pass(cidx_hbm.at[wid], ones_hbm)
    plsc.subcore_barrier()
    pltpu.sync_copy(acc_sh.at[pl.ds(r0, ROWS_PER_TILE)],
                    cnt_hbm.at[c, pl.ds(r0, ROWS_PER_TILE)])
    zero_my_rows()
    plsc.subcore_barrier()

    def t_body(t, carry):
        run_pass(gidx_hbm.at[t, wid], x_hbm)
        plsc.subcore_barrier()
        pltpu.sync_copy(acc_sh.at[pl.ds(r0, ROWS_PER_TILE)],
                        agg_hbm.at[c, t, pl.ds(r0, ROWS_PER_TILE)])
        zero_my_rows()
        plsc.subcore_barrier()
        return carry
    lax.fori_loop(0, T_STEPS, t_body, 0)


@jax.jit
def _sc_aggregate(x_flat, gidx, cidx, dst, zeros, ones):
    mesh = plsc.VectorSubcoreMesh(core_axis_name="c", subcore_axis_name="s")
    run = pl.kernel(
        _sc_agg_body,
        out_type=[
            jax.ShapeDtypeStruct((NUM_CORES, T_STEPS, N_PAD, C_IN),
                                 jnp.float32),
            jax.ShapeDtypeStruct((NUM_CORES, N_PAD, C_IN), jnp.float32),
        ],
        mesh=mesh,
        scratch_types=[
            pltpu.VMEM((GRP_CHUNKS, CHUNK), jnp.int32),
            pltpu.VMEM((GRP_CHUNKS, CHUNK), jnp.int32),
            pltpu.VMEM((CHUNK, C_IN), jnp.float32),
            pltpu.VMEM((CHUNK, C_IN), jnp.float32),
            pltpu.VMEM((CHUNK, C_IN), jnp.float32),
            pltpu.VMEM_SHARED((N_PAD, C_IN), jnp.float32),
            pltpu.SemaphoreType.DMA,
            pltpu.SemaphoreType.DMA,
            pltpu.SemaphoreType.DMA,
        ],
    )
    return run(x_flat, gidx, cidx, dst, zeros, ones)


def _tc_dense_body(x_ref, agg_ref, cnt_ref, wsage_ref, bl_ref,
                   wrz_ref, brz_ref, win_ref, bin_ref, whn_ref, bhn_ref,
                   whead_ref, bhead_ref,
                   logits_ref, recon_ref):
    cnt = cnt_ref[0, :, 0:1] + cnt_ref[1, :, 0:1]
    rdenom = 1.0 / jnp.maximum(cnt, 1.0)
    nb = x_ref.shape[0]
    h = jnp.zeros((nb, HID), jnp.float32)
    for t in range(T_STEPS):
        mean = (agg_ref[0, t] + agg_ref[1, t]) * rdenom
        xmean = jnp.concatenate([mean, x_ref[:, t, :]], axis=1)
        sp = jnp.maximum(
            jnp.dot(xmean, wsage_ref[...],
                    preferred_element_type=jnp.float32) + bl_ref[...], 0.0)
        sph = jnp.concatenate([sp, h], axis=1)
        rz = jnp.dot(sph, wrz_ref[...],
                     preferred_element_type=jnp.float32) + brz_ref[...]
        r = jax.nn.sigmoid(rz[:, :HID])
        z = jax.nn.sigmoid(rz[:, HID:])
        i_n = jnp.dot(sp, win_ref[...],
                      preferred_element_type=jnp.float32) + bin_ref[...]
        h_n = jnp.dot(h, whn_ref[...],
                      preferred_element_type=jnp.float32) + bhn_ref[...]
        ng = jnp.tanh(i_n + r * h_n)
        h = (1.0 - z) * ng + z * h
        heads = jnp.dot(h, whead_ref[...],
                        preferred_element_type=jnp.float32) + bhead_ref[...]
        logits_ref[:, t, :] = heads[:, :HID]
        recon_ref[:, t, :] = heads[:, HID:]


@functools.partial(jax.jit, static_argnames=("nb", "interpret"))
def _tc_dense(x, agg, cnt, wsage, bl2, wrz, brz, win, bin2, whn, bhn,
              whead, bhead, nb=400, interpret=False):
    n = x.shape[0]
    grid = (n // nb,)
    full = lambda shape: pl.BlockSpec(shape, lambda i: tuple(0 for _ in shape))
    return pl.pallas_call(
        _tc_dense_body,
        grid=grid,
        in_specs=[
            pl.BlockSpec((nb, T_STEPS, C_IN), lambda i: (i, 0, 0)),
            pl.BlockSpec((NUM_CORES, T_STEPS, nb, C_IN),
                         lambda i: (0, 0, i, 0)),
            pl.BlockSpec((NUM_CORES, nb, C_IN), lambda i: (0, i, 0)),
            full((2 * C_IN, HID)), full((1, HID)),
            full((2 * HID, 2 * HID)), full((1, 2 * HID)),
            full((HID, HID)), full((1, HID)),
            full((HID, HID)), full((1, HID)),
            full((HID, 2 * HID)), full((1, 2 * HID)),
        ],
        out_specs=[
            pl.BlockSpec((nb, T_STEPS, HID), lambda i: (i, 0, 0)),
            pl.BlockSpec((nb, T_STEPS, C_IN), lambda i: (i, 0, 0)),
        ],
        out_shape=[
            jax.ShapeDtypeStruct((n, T_STEPS, HID), jnp.float32),
            jax.ShapeDtypeStruct((n, T_STEPS, C_IN), jnp.float32),
        ],
        interpret=interpret,
    )(x, agg, cnt, wsage, bl2, wrz, brz, win, bin2, whn, bhn, whead, bhead)


def kernel(x, edge_index, Wl, bl, Wr, W_ih, W_hh, b_ih, b_hh,
           W_cls, b_cls, W_rec, b_rec):
    src = edge_index[0]
    dst = edge_index[1]
    gidx = (src[None, :] * T_STEPS
            + jnp.arange(T_STEPS, dtype=jnp.int32)[:, None])
    gidx = gidx.reshape(T_STEPS, NUM_WORKERS, NUM_GRPS, GRP_CHUNKS, CHUNK)
    cidx = jnp.remainder(jnp.arange(N_EDGES, dtype=jnp.int32), 128)
    cidx = cidx.reshape(NUM_WORKERS, NUM_GRPS, GRP_CHUNKS, CHUNK)
    dst_r = dst.reshape(NUM_WORKERS, NUM_GRPS, GRP_CHUNKS, CHUNK)
    x_flat = x.reshape(N_NODES * T_STEPS, C_IN)
    zeros = jnp.zeros((ROWS_PER_TILE, C_IN), jnp.float32)
    ones = jnp.ones((128, C_IN), jnp.float32)
    agg, cnt = _sc_aggregate(x_flat, gidx, cidx, dst_r, zeros, ones)
    wsage = jnp.concatenate([Wl.T, Wr.T], axis=0)
    wrz = jnp.concatenate(
        [jnp.concatenate([W_ih[:HID].T, W_ih[HID:2 * HID].T], axis=1),
         jnp.concatenate([W_hh[:HID].T, W_hh[HID:2 * HID].T], axis=1)],
        axis=0)
    brz = jnp.concatenate(
        [b_ih[:HID] + b_hh[:HID], b_ih[HID:2 * HID] + b_hh[HID:2 * HID]])
    win = W_ih[2 * HID:].T
    bin2 = b_ih[2 * HID:]
    whn = W_hh[2 * HID:].T
    bhn = b_hh[2 * HID:]
    wcls_pad = jnp.zeros((HID, HID), jnp.float32).at[:, :C_CLS].set(W_cls.T)
    bcls_pad = jnp.zeros((HID,), jnp.float32).at[:C_CLS].set(b_cls)
    whead = jnp.concatenate([wcls_pad, W_rec.T], axis=1)
    bhead = jnp.concatenate([bcls_pad, b_rec])
    logits_pad, recon = _tc_dense(
        x, agg, cnt, wsage, bl[None, :], wrz, brz[None, :],
        win, bin2[None, :], whn, bhn[None, :], whead, bhead[None, :])
    return (logits_pad[:, :, :C_CLS], recon)


# speedup vs baseline: 1.1085x; 1.0658x over previous
import functools

import jax
import jax.numpy as jnp
from jax import lax
from jax.experimental import pallas as pl
from jax.experimental.pallas import tpu as pltpu
from jax.experimental.pallas import tpu_sc as plsc

N_NODES = 10000
T_STEPS = 12
C_IN = 128
HID = 128
C_CLS = 10
N_EDGES = 320000

NUM_CORES = 2
NUM_SUBCORES = 16
NUM_WORKERS = NUM_CORES * NUM_SUBCORES
EDGES_PER_WORKER = N_EDGES // NUM_WORKERS
CHUNK = 80
NUM_CHUNKS = EDGES_PER_WORKER // CHUNK
NUM_GRPS = 5
GRP_CHUNKS = NUM_CHUNKS // NUM_GRPS
N_PAD = 10240
ROWS_PER_TILE = N_PAD // NUM_SUBCORES
ZROWS = 64


def _sc_agg_body(x_hbm, gidx_hbm, cidx_hbm, dst_hbm, zeros_hbm, ones_hbm,
                 agg_hbm, cnt_hbm,
                 gidx_v, dst_v, rows_a, rows_b, rows_c, acc_sh,
                 sem_a, sem_b, sem_c):
    c = lax.axis_index("c")
    s = lax.axis_index("s")
    wid = s * NUM_CORES + c
    r0 = s * ROWS_PER_TILE

    def zero_my_rows():
        pltpu.sync_copy(zeros_hbm, acc_sh.at[pl.ds(r0, ROWS_PER_TILE)])

    zero_my_rows()
    plsc.subcore_barrier()

    def run_pass(idx_hbm_slab, table_hbm):
        for g in range(NUM_GRPS):
            pltpu.sync_copy(idx_hbm_slab.at[g], gidx_v)
            pltpu.sync_copy(dst_hbm.at[wid, g], dst_v)
            pltpu.async_copy(table_hbm.at[gidx_v.at[0]], rows_a, sem_a)
            pltpu.async_copy(table_hbm.at[gidx_v.at[1]], rows_b, sem_b)

            def tri_body(p, carry):
                b = 3 * p
                pltpu.async_copy(table_hbm.at[gidx_v.at[b + 2]],
                                 rows_c, sem_c)
                pltpu.make_async_copy(table_hbm.at[gidx_v.at[b]],
                                      rows_a, sem_a).wait()
                pltpu.sync_copy(rows_a, acc_sh.at[dst_v.at[b]], add=True)
                pltpu.async_copy(table_hbm.at[gidx_v.at[b + 3]],
                                 rows_a, sem_a)
                pltpu.make_async_copy(table_hbm.at[gidx_v.at[b + 1]],
                                      rows_b, sem_b).wait()
                pltpu.sync_copy(rows_b, acc_sh.at[dst_v.at[b + 1]], add=True)
                pltpu.async_copy(table_hbm.at[gidx_v.at[b + 4]],
                                 rows_b, sem_b)
                pltpu.make_async_copy(table_hbm.at[gidx_v.at[b + 2]],
                                      rows_c, sem_c).wait()
                pltpu.sync_copy(rows_c, acc_sh.at[dst_v.at[b + 2]], add=True)
                return carry
            lax.fori_loop(0, 7, tri_body, 0)
            pltpu.async_copy(table_hbm.at[gidx_v.at[23]], rows_c, sem_c)
            pltpu.make_async_copy(table_hbm.at[gidx_v.at[21]],
                                  rows_a, sem_a).wait()
            pltpu.sync_copy(rows_a, acc_sh.at[dst_v.at[21]], add=True)
            pltpu.async_copy(table_hbm.at[gidx_v.at[24]], rows_a, sem_a)
            pltpu.make_async_copy(table_hbm.at[gidx_v.at[22]],
                                  rows_b, sem_b).wait()
            pltpu.sync_copy(rows_b, acc_sh.at[dst_v.at[22]], add=True)
            pltpu.make_async_copy(table_hbm.at[gidx_v.at[23]],
                                  rows_c, sem_c).wait()
            pltpu.sync_copy(rows_c, acc_sh.at[dst_v.at[23]], add=True)
            pltpu.make_async_copy(table_hbm.at[gidx_v.at[24]],
                                  rows_a, sem_a).wait()
            pltpu.sync_copy(rows_a, acc_sh.at[dst_v.at[24]], add=True)

    run_pass(cidx_hbm.at[wid], ones_hbm)
    plsc.subcore_barrier()
    pltpu.sync_copy(acc_sh.at[pl.ds(r0, ROWS_PER_TILE)],
                    cnt_hbm.at[c, pl.ds(r0, ROWS_PER_TILE)])
    zero_my_rows()
    plsc.subcore_barrier()

    def t_body(t, carry):
        run_pass(gidx_hbm.at[t, wid], x_hbm)
        plsc.subcore_barrier()
        pltpu.sync_copy(acc_sh.at[pl.ds(r0, ROWS_PER_TILE)],
                        agg_hbm.at[c, t, pl.ds(r0, ROWS_PER_TILE)])
        zero_my_rows()
        plsc.subcore_barrier()
        return carry
    lax.fori_loop(0, T_STEPS, t_body, 0)


@jax.jit
def _sc_aggregate(x_flat, gidx, cidx, dst, zeros, ones):
    mesh = plsc.VectorSubcoreMesh(core_axis_name="c", subcore_axis_name="s")
    run = pl.kernel(
        _sc_agg_body,
        out_type=[
            jax.ShapeDtypeStruct((NUM_CORES, T_STEPS, N_PAD, C_IN),
                                 jnp.float32),
            jax.ShapeDtypeStruct((NUM_CORES, N_PAD, C_IN), jnp.float32),
        ],
        mesh=mesh,
        scratch_types=[
            pltpu.VMEM((GRP_CHUNKS, CHUNK), jnp.int32),
            pltpu.VMEM((GRP_CHUNKS, CHUNK), jnp.int32),
            pltpu.VMEM((CHUNK, C_IN), jnp.float32),
            pltpu.VMEM((CHUNK, C_IN), jnp.float32),
            pltpu.VMEM((CHUNK, C_IN), jnp.float32),
            pltpu.VMEM_SHARED((N_PAD, C_IN), jnp.float32),
            pltpu.SemaphoreType.DMA,
            pltpu.SemaphoreType.DMA,
            pltpu.SemaphoreType.DMA,
        ],
    )
    return run(x_flat, gidx, cidx, dst, zeros, ones)


def _tc_dense_body(x_ref, agg_ref, cnt_ref, wsage_ref, bl_ref,
                   wrz_ref, brz_ref, win_ref, bin_ref, whn_ref, bhn_ref,
                   whead_ref, bhead_ref,
                   logits_ref, recon_ref):
    cnt = cnt_ref[0, :, 0:1] + cnt_ref[1, :, 0:1]
    rdenom = 1.0 / jnp.maximum(cnt, 1.0)
    nb = x_ref.shape[0]
    h = jnp.zeros((nb, HID), jnp.float32)
    for t in range(T_STEPS):
        mean = (agg_ref[0, t] + agg_ref[1, t]) * rdenom
        xmean = jnp.concatenate([mean, x_ref[:, t, :]], axis=1)
        sp = jnp.maximum(
            jnp.dot(xmean, wsage_ref[...],
                    preferred_element_type=jnp.float32) + bl_ref[...], 0.0)
        sph = jnp.concatenate([sp, h], axis=1)
        rz = jnp.dot(sph, wrz_ref[...],
                     preferred_element_type=jnp.float32) + brz_ref[...]
        r = jax.nn.sigmoid(rz[:, :HID])
        z = jax.nn.sigmoid(rz[:, HID:])
        i_n = jnp.dot(sp, win_ref[...],
                      preferred_element_type=jnp.float32) + bin_ref[...]
        h_n = jnp.dot(h, whn_ref[...],
                      preferred_element_type=jnp.float32) + bhn_ref[...]
        ng = jnp.tanh(i_n + r * h_n)
        h = (1.0 - z) * ng + z * h
        heads = jnp.dot(h, whead_ref[...],
                        preferred_element_type=jnp.float32) + bhead_ref[...]
        logits_ref[:, t, :] = heads[:, :HID]
        recon_ref[:, t, :] = heads[:, HID:]


@functools.partial(jax.jit, static_argnames=("nb", "interpret"))
def _tc_dense(x, agg, cnt, wsage, bl2, wrz, brz, win, bin2, whn, bhn,
              whead, bhead, nb=400, interpret=False):
    n = x.shape[0]
    grid = (n // nb,)
    full = lambda shape: pl.BlockSpec(shape, lambda i: tuple(0 for _ in shape))
    return pl.pallas_call(
        _tc_dense_body,
        grid=grid,
        in_specs=[
            pl.BlockSpec((nb, T_STEPS, C_IN), lambda i: (i, 0, 0)),
            pl.BlockSpec((NUM_CORES, T_STEPS, nb, C_IN),
                         lambda i: (0, 0, i, 0)),
            pl.BlockSpec((NUM_CORES, nb, C_IN), lambda i: (0, i, 0)),
            full((2 * C_IN, HID)), full((1, HID)),
            full((2 * HID, 2 * HID)), full((1, 2 * HID)),
            full((HID, HID)), full((1, HID)),
            full((HID, HID)), full((1, HID)),
            full((HID, 2 * HID)), full((1, 2 * HID)),
        ],
        out_specs=[
            pl.BlockSpec((nb, T_STEPS, HID), lambda i: (i, 0, 0)),
            pl.BlockSpec((nb, T_STEPS, C_IN), lambda i: (i, 0, 0)),
        ],
        out_shape=[
            jax.ShapeDtypeStruct((n, T_STEPS, HID), jnp.float32),
            jax.ShapeDtypeStruct((n, T_STEPS, C_IN), jnp.float32),
        ],
        interpret=interpret,
    )(x, agg, cnt, wsage, bl2, wrz, brz, win, bin2, whn, bhn, whead, bhead)


def kernel(x, edge_index, Wl, bl, Wr, W_ih, W_hh, b_ih, b_hh,
           W_cls, b_cls, W_rec, b_rec):
    src = edge_index[0]
    dst = edge_index[1]
    gidx = (src[None, :] * T_STEPS
            + jnp.arange(T_STEPS, dtype=jnp.int32)[:, None])
    gidx = gidx.reshape(T_STEPS, NUM_WORKERS, NUM_GRPS, GRP_CHUNKS, CHUNK)
    cidx = jnp.remainder(jnp.arange(N_EDGES, dtype=jnp.int32), 4096)
    cidx = cidx.reshape(NUM_WORKERS, NUM_GRPS, GRP_CHUNKS, CHUNK)
    dst_r = dst.reshape(NUM_WORKERS, NUM_GRPS, GRP_CHUNKS, CHUNK)
    x_flat = x.reshape(N_NODES * T_STEPS, C_IN)
    zeros = jnp.zeros((ROWS_PER_TILE, C_IN), jnp.float32)
    ones = jnp.ones((4096, C_IN), jnp.float32)
    agg, cnt = _sc_aggregate(x_flat, gidx, cidx, dst_r, zeros, ones)
    wsage = jnp.concatenate([Wl.T, Wr.T], axis=0)
    wrz = jnp.concatenate(
        [jnp.concatenate([W_ih[:HID].T, W_ih[HID:2 * HID].T], axis=1),
         jnp.concatenate([W_hh[:HID].T, W_hh[HID:2 * HID].T], axis=1)],
        axis=0)
    brz = jnp.concatenate(
        [b_ih[:HID] + b_hh[:HID], b_ih[HID:2 * HID] + b_hh[HID:2 * HID]])
    win = W_ih[2 * HID:].T
    bin2 = b_ih[2 * HID:]
    whn = W_hh[2 * HID:].T
    bhn = b_hh[2 * HID:]
    wcls_pad = jnp.zeros((HID, HID), jnp.float32).at[:, :C_CLS].set(W_cls.T)
    bcls_pad = jnp.zeros((HID,), jnp.float32).at[:C_CLS].set(b_cls)
    whead = jnp.concatenate([wcls_pad, W_rec.T], axis=1)
    bhead = jnp.concatenate([bcls_pad, b_rec])
    logits_pad, recon = _tc_dense(
        x, agg, cnt, wsage, bl[None, :], wrz, brz[None, :],
        win, bin2[None, :], whn, bhn[None, :], whead, bhead[None, :])
    return (logits_pad[:, :, :C_CLS], recon)
